# Initial kernel scaffold; baseline (speedup 1.0000x reference)
import jax
import jax.numpy as jnp
from jax.experimental import pallas as pl


def kernel(x, edge_index, Ws1, Wn1, b1, Ws2, Wn2, b2, Ws3, Wn3, b3):
    raise NotImplementedError("write your pallas kernel here")


# speedup vs baseline: 3.8834x; 3.8834x over previous
import functools

import jax
import jax.numpy as jnp
from jax import lax
from jax.experimental import pallas as pl
from jax.experimental.pallas import tpu as pltpu
from jax.experimental.pallas import tpu_sc as plsc

N = 10000
E = 320000
C = 128
E_PAD = 32 * 79 * C
PAD = E_PAD - E
N_PAD = 10240
ZR = N_PAD // 16
EPW16 = E_PAD // 16
CH16 = EPW16 // C
EPW32 = E_PAD // 32
CH32 = EPW32 // C
RT = 1000


def _make_agg_edge_split():
    mesh = plsc.VectorSubcoreMesh(core_axis_name="c", subcore_axis_name="s")
    out_t = jax.ShapeDtypeStruct((2 * N_PAD, 128), jnp.float32)
    scratch = [
        pltpu.VMEM((C,), jnp.int32),
        pltpu.VMEM((C,), jnp.int32),
        pltpu.VMEM((C, 128), jnp.float32),
        pltpu.VMEM_SHARED((N_PAD, 128), jnp.float32),
        pltpu.SemaphoreType.DMA,
    ]

    @functools.partial(pl.kernel, mesh=mesh, out_type=out_t,
                       scratch_types=scratch)
    def k(xsrc, src_p, dst_p, zrows, out, sidx, didx, rows, acc, sem):
        cid = lax.axis_index("c")
        sid = lax.axis_index("s")
        pltpu.sync_copy(zrows, acc.at[pl.ds(sid * ZR, ZR)])
        plsc.subcore_barrier()
        ebase = (cid * 16 + sid) * EPW32

        @pl.loop(0, CH32)
        def _edges(t):
            base = ebase + t * C
            pltpu.sync_copy(src_p.at[pl.ds(base, C)], sidx)
            pltpu.sync_copy(dst_p.at[pl.ds(base, C)], didx)
            pltpu.async_copy(xsrc.at[sidx], rows, sem).wait()
            pltpu.sync_copy(rows, acc.at[didx], add=True)

        plsc.subcore_barrier()
        pltpu.sync_copy(acc.at[pl.ds(sid * ZR, ZR)],
                        out.at[pl.ds(cid * N_PAD + sid * ZR, ZR)])

    return k


def _make_agg_feat_split():
    mesh = plsc.VectorSubcoreMesh(core_axis_name="c", subcore_axis_name="s")
    out_t = jax.ShapeDtypeStruct((2 * N_PAD, 128), jnp.float32)
    scratch = [
        pltpu.VMEM((C,), jnp.int32),
        pltpu.VMEM((C,), jnp.int32),
        pltpu.VMEM((C, 128), jnp.float32),
        pltpu.VMEM_SHARED((N_PAD, 128), jnp.float32),
        pltpu.SemaphoreType.DMA,
    ]

    @functools.partial(pl.kernel, mesh=mesh, out_type=out_t,
                       scratch_types=scratch)
    def k(xcat, src2, dst_p, zrows, out, sidx, didx, rows, acc, sem):
        cid = lax.axis_index("c")
        sid = lax.axis_index("s")
        pltpu.sync_copy(zrows, acc.at[pl.ds(sid * ZR, ZR)])
        plsc.subcore_barrier()
        ebase = cid * E_PAD + sid * EPW16

        @pl.loop(0, CH16)
        def _edges(t):
            base = ebase + t * C
            pltpu.sync_copy(src2.at[pl.ds(base, C)], sidx)
            pltpu.sync_copy(dst_p.at[pl.ds(sid * EPW16 + t * C, C)], didx)
            pltpu.async_copy(xcat.at[sidx], rows, sem).wait()
            pltpu.sync_copy(rows, acc.at[didx], add=True)

        plsc.subcore_barrier()
        pltpu.sync_copy(acc.at[pl.ds(sid * ZR, ZR)],
                        out.at[pl.ds(cid * N_PAD + sid * ZR, ZR)])

    return k


DW = 128


def _make_deg():
    mesh = plsc.VectorSubcoreMesh(core_axis_name="c", subcore_axis_name="s")
    out_t = jax.ShapeDtypeStruct((2 * N_PAD, DW), jnp.float32)
    scratch = [
        pltpu.VMEM((C,), jnp.int32),
        pltpu.VMEM((C, DW), jnp.float32),
        pltpu.VMEM_SHARED((N_PAD, DW), jnp.float32),
    ]

    @functools.partial(pl.kernel, mesh=mesh, out_type=out_t,
                       scratch_types=scratch)
    def k(dst_p, ones_h, zrows, out, didx, onesv, acc):
        cid = lax.axis_index("c")
        sid = lax.axis_index("s")
        pltpu.sync_copy(zrows, acc.at[pl.ds(sid * ZR, ZR)])
        pltpu.sync_copy(ones_h, onesv)
        plsc.subcore_barrier()
        ebase = (cid * 16 + sid) * EPW32

        @pl.loop(0, CH32)
        def _edges(t):
            base = ebase + t * C
            pltpu.sync_copy(dst_p.at[pl.ds(base, C)], didx)
            pltpu.sync_copy(onesv, acc.at[didx], add=True)

        plsc.subcore_barrier()
        pltpu.sync_copy(acc.at[pl.ds(sid * ZR, ZR)],
                        out.at[pl.ds(cid * N_PAD + sid * ZR, ZR)])

    return k


_DEG = _make_deg()
_AGG_ES = _make_agg_edge_split()
_AGG_FS = _make_agg_feat_split()


def _tc_layer(x, a0, a1, d0, d1, Ws, Wn0, Wn1, b, feat_split, Wx=None):
    K = x.shape[1]
    M = Ws.shape[1]
    fc = a0.shape[1]
    has_x = Wx is not None

    def body(x_r, a0_r, a1_r, d0_r, d1_r, ws_r, wn0_r, wn1_r, b_r, *rest):
        if has_x:
            wx_r, h_r, hx_r = rest
        else:
            (h_r,) = rest
        deg = d0_r[:, 0:1] + d1_r[:, 0:1]
        inv = 1.0 / jnp.maximum(deg, 1.0)
        acc = jnp.dot(x_r[...], ws_r[...], preferred_element_type=jnp.float32)
        if feat_split:
            acc += jnp.dot(a0_r[...] * inv, wn0_r[...],
                           preferred_element_type=jnp.float32)
            acc += jnp.dot(a1_r[...] * inv, wn1_r[...],
                           preferred_element_type=jnp.float32)
        else:
            acc += jnp.dot((a0_r[...] + a1_r[...]) * inv, wn0_r[...],
                           preferred_element_type=jnp.float32)
        h = jnp.maximum(acc + b_r[...], 0.0)
        h_r[...] = h
        if has_x:
            hx_r[...] = jnp.dot(h, wx_r[...],
                                preferred_element_type=jnp.float32)

    row = lambda i: (i, 0)
    rep = lambda i: (0, 0)
    in_specs = [
        pl.BlockSpec((RT, K), row),
        pl.BlockSpec((RT, fc), row),
        pl.BlockSpec((RT, fc), row),
        pl.BlockSpec((RT, DW), row),
        pl.BlockSpec((RT, DW), row),
        pl.BlockSpec((K, M), rep),
        pl.BlockSpec((fc, M), rep),
        pl.BlockSpec((fc, M), rep),
        pl.BlockSpec((1, M), rep),
    ]
    out_shape = [jax.ShapeDtypeStruct((N, M), jnp.float32)]
    out_specs = [pl.BlockSpec((RT, M), row)]
    args = [x, a0, a1, d0, d1, Ws, Wn0, Wn1, b]
    if has_x:
        MX = Wx.shape[1]
        in_specs.append(pl.BlockSpec((M, MX), rep))
        out_shape.append(jax.ShapeDtypeStruct((N, MX), jnp.float32))
        out_specs.append(pl.BlockSpec((RT, MX), row))
        args.append(Wx)
    res = pl.pallas_call(
        body, grid=(N // RT,), in_specs=in_specs, out_specs=out_specs,
        out_shape=out_shape)(*args)
    return res if has_x else res[0]


def _tc_final(h, a0, a1, d0, d1, Ws, b):
    K = h.shape[1]
    M = Ws.shape[1]

    def body(h_r, a0_r, a1_r, d0_r, d1_r, ws_r, b_r, o_r):
        deg = d0_r[:, 0:1] + d1_r[:, 0:1]
        inv = 1.0 / jnp.maximum(deg, 1.0)
        acc = jnp.dot(h_r[...], ws_r[...], preferred_element_type=jnp.float32)
        o_r[...] = acc + (a0_r[...] + a1_r[...]) * inv + b_r[...]

    row = lambda i: (i, 0)
    rep = lambda i: (0, 0)
    return pl.pallas_call(
        body, grid=(N // RT,),
        in_specs=[
            pl.BlockSpec((RT, K), row),
            pl.BlockSpec((RT, M), row),
            pl.BlockSpec((RT, M), row),
            pl.BlockSpec((RT, DW), row),
            pl.BlockSpec((RT, DW), row),
            pl.BlockSpec((K, M), rep),
            pl.BlockSpec((1, M), rep),
        ],
        out_specs=pl.BlockSpec((RT, M), row),
        out_shape=jax.ShapeDtypeStruct((N, M), jnp.float32),
    )(h, a0, a1, d0, d1, Ws, b)


def kernel(x, edge_index, Ws1, Wn1, b1, Ws2, Wn2, b2, Ws3, Wn3, b3):
    src = edge_index[0].astype(jnp.int32)
    dst = edge_index[1].astype(jnp.int32)
    src_p = jnp.concatenate([src, jnp.zeros((PAD,), jnp.int32)])
    dst_p = jnp.concatenate([dst, jnp.full((PAD,), N, jnp.int32)])
    src2 = jnp.concatenate([src_p, src_p + N])
    ones8 = jnp.ones((C, DW), jnp.float32)
    z8 = jnp.zeros((ZR, DW), jnp.float32)
    z128 = jnp.zeros((ZR, 128), jnp.float32)

    d = _DEG(dst_p, ones8, z8)
    d0, d1 = d[:N_PAD], d[N_PAD:]

    a = _AGG_ES(x, src_p, dst_p, z128)
    h1 = _tc_layer(x, a[:N_PAD], a[N_PAD:], d0, d1, Ws1, Wn1, Wn1,
                   b1[None, :], feat_split=False)

    h1cat = jnp.concatenate([h1[:, :128], h1[:, 128:]], axis=0)
    a = _AGG_FS(h1cat, src2, dst_p, z128)
    h2, xn3 = _tc_layer(h1, a[:N_PAD], a[N_PAD:], d0, d1, Ws2, Wn2[:128],
                        Wn2[128:], b2[None, :], feat_split=True, Wx=Wn3)

    a = _AGG_ES(xn3, src_p, dst_p, z128)
    return _tc_final(h2, a[:N_PAD], a[N_PAD:], d0, d1, Ws3, b3[None, :])
